# baseline (speedup 1.0000x reference)
import functools

import jax
import jax.numpy as jnp
from jax.experimental import pallas as pl
from jax.experimental.pallas import tpu as pltpu

_BN_EPS = 1e-5
_ALIGN = 128


def _conv_stats_kernel(x_ref, w_ref, y_ref, stats_ref, xs_ref, *, h, w):
    n_pix = h * w
    halo = w + 1
    cin = x_ref.shape[1]
    xs_ref[:, :_ALIGN] = jnp.zeros((cin, _ALIGN), xs_ref.dtype)
    xs_ref[:, _ALIGN + n_pix:] = jnp.zeros(
        (cin, xs_ref.shape[1] - _ALIGN - n_pix), xs_ref.dtype
    )
    xs_ref[:, _ALIGN:_ALIGN + n_pix] = x_ref[0].astype(xs_ref.dtype)
    xs = xs_ref[...]

    accs = []
    for kw in range(3):
        acc = jnp.zeros((y_ref.shape[1], n_pix), jnp.float32)
        for kh in range(3):
            off = _ALIGN - halo + kh * w + kw
            acc = acc + jnp.dot(
                w_ref[3 * kh + kw], xs[:, off:off + n_pix],
                preferred_element_type=jnp.float32,
            )
        accs.append(acc)
    col = jax.lax.broadcasted_iota(jnp.int32, (1, n_pix), 1) % w
    m0 = (col > 0).astype(jnp.float32)
    m2 = (col < w - 1).astype(jnp.float32)
    acc = accs[1] + m0 * accs[0] + m2 * accs[2]

    y_ref[0] = acc.astype(y_ref.dtype)
    ssum = jnp.sum(acc, axis=1, keepdims=True)
    ssq = jnp.sum(acc * acc, axis=1, keepdims=True)
    stats_ref[0] = jnp.concatenate(
        [ssum, ssq] + [jnp.zeros_like(ssum)] * 6, axis=1
    )


def _bn_relu_kernel(y_ref, stats_ref, g_ref, b_ref, out_ref, *, n_valid):
    st = stats_ref[...]
    ssum = jnp.sum(st[:, :, 0], axis=0)
    ssq = jnp.sum(st[:, :, 1], axis=0)
    mean = ssum / n_valid
    var = jnp.maximum(ssq / n_valid - mean * mean, 0.0)
    scale = g_ref[...][:, 0] * jax.lax.rsqrt(var + _BN_EPS)
    shift = b_ref[...][:, 0] - mean * scale
    y = y_ref[0].astype(jnp.float32)
    out_ref[0] = jnp.maximum(
        y * scale.reshape(-1, 1) + shift.reshape(-1, 1), 0.0
    )


def kernel(x_nchw, w_oihw, bias, gamma, beta):
    del bias
    N, C, H, W = x_nchw.shape
    Cout, _, KH, KW = w_oihw.shape
    assert KH == 3 and KW == 3

    P = H * W
    NC = 2
    NI = N // NC

    x_flat = x_nchw.reshape(N, C, P)
    wt = jnp.transpose(w_oihw, (2, 3, 0, 1)).reshape(9, Cout, C)
    wt = wt.astype(jnp.bfloat16)

    cparams = pltpu.CompilerParams(
        dimension_semantics=("parallel", "arbitrary"),
        vmem_limit_bytes=96 * 1024 * 1024,
    )

    y, stats = pl.pallas_call(
        functools.partial(_conv_stats_kernel, h=H, w=W),
        out_shape=(
            jax.ShapeDtypeStruct((N, Cout, P), jnp.bfloat16),
            jax.ShapeDtypeStruct((N, Cout, 8), jnp.float32),
        ),
        grid=(NC, NI),
        in_specs=[
            pl.BlockSpec((1, C, P), lambda c, j: (c * NI + j, 0, 0)),
            pl.BlockSpec((9, Cout, C), lambda c, j: (0, 0, 0)),
        ],
        out_specs=(
            pl.BlockSpec((1, Cout, P), lambda c, j: (c * NI + j, 0, 0)),
            pl.BlockSpec((1, Cout, 8), lambda c, j: (c * NI + j, 0, 0)),
        ),
        scratch_shapes=[pltpu.VMEM((C, _ALIGN + P + _ALIGN), jnp.bfloat16)],
        compiler_params=cparams,
    )(x_flat, wt)

    out_flat = pl.pallas_call(
        functools.partial(_bn_relu_kernel, n_valid=N * P),
        out_shape=jax.ShapeDtypeStruct((N, Cout, P), jnp.float32),
        grid=(NC, NI),
        in_specs=[
            pl.BlockSpec((1, Cout, P), lambda c, j: (c * NI + j, 0, 0)),
            pl.BlockSpec((N, Cout, 8), lambda c, j: (0, 0, 0)),
            pl.BlockSpec((Cout, 1), lambda c, j: (0, 0)),
            pl.BlockSpec((Cout, 1), lambda c, j: (0, 0)),
        ],
        out_specs=pl.BlockSpec((1, Cout, P), lambda c, j: (c * NI + j, 0, 0)),
        compiler_params=cparams,
    )(y, stats, gamma.astype(jnp.float32).reshape(Cout, 1),
      beta.astype(jnp.float32).reshape(Cout, 1))
    return out_flat.reshape(N, Cout, H, W)


# speedup vs baseline: 1.1645x; 1.1645x over previous
import functools

import jax
import jax.numpy as jnp
from jax.experimental import pallas as pl
from jax.experimental.pallas import tpu as pltpu

_BN_EPS = 1e-5
_ALIGN = 128
_BATCH = 2


def _fused_kernel(x_ref, w_ref, g_ref, b_ref, out_ref, xs_ref, ys_ref,
                  st_ref, mask_ref, *, h, w, n):
    n_pix = h * w
    halo = w + 1
    phase = pl.program_id(0)
    step = pl.program_id(1)

    @pl.when((phase == 0) & (step == 0))
    def _init_constants():
        cin = x_ref.shape[1]
        xs_ref[:, :_ALIGN] = jnp.zeros((cin, _ALIGN), xs_ref.dtype)
        xs_ref[:, _ALIGN + n_pix:] = jnp.zeros(
            (cin, xs_ref.shape[1] - _ALIGN - n_pix), xs_ref.dtype
        )
        col = jax.lax.broadcasted_iota(jnp.int32, (8, n_pix), 1) % w
        mask_ref[0:8] = (col > 0).astype(jnp.float32)
        mask_ref[8:16] = (col < w - 1).astype(jnp.float32)

    @pl.when(phase == 0)
    def _conv_phase():
        m0 = mask_ref[0:1]
        m2 = mask_ref[8:9]
        part = jnp.zeros((st_ref.shape[0], 2), jnp.float32)
        for j in range(_BATCH):
            xs_ref[:, _ALIGN:_ALIGN + n_pix] = x_ref[j].astype(xs_ref.dtype)
            xs = xs_ref[...]
            accs = []
            for kw in range(3):
                acc = jnp.zeros((ys_ref.shape[1], n_pix), jnp.float32)
                for kh in range(3):
                    off = _ALIGN - halo + kh * w + kw
                    acc = acc + jnp.dot(
                        w_ref[3 * kh + kw], xs[:, off:off + n_pix],
                        preferred_element_type=jnp.float32,
                    )
                accs.append(acc)
            acc = accs[1] + m0 * accs[0] + m2 * accs[2]
            ys_ref[step * _BATCH + j] = acc.astype(ys_ref.dtype)
            ssum = jnp.sum(acc, axis=1, keepdims=True)
            ssq = jnp.sum(acc * acc, axis=1, keepdims=True)
            part = part + jnp.concatenate([ssum, ssq], axis=1)

        @pl.when(step == 0)
        def _init():
            st_ref[...] = part

        @pl.when(step > 0)
        def _accum():
            st_ref[...] = st_ref[...] + part

    @pl.when(phase == 1)
    def _bn_phase():
        st = st_ref[...]
        mean = st[:, 0] / n
        var = jnp.maximum(st[:, 1] / n - mean * mean, 0.0)
        scale = g_ref[...][:, 0] * jax.lax.rsqrt(var + _BN_EPS)
        shift = b_ref[...][:, 0] - mean * scale
        for j in range(_BATCH):
            y = ys_ref[step * _BATCH + j].astype(jnp.float32)
            out_ref[j] = jnp.maximum(
                y * scale.reshape(-1, 1) + shift.reshape(-1, 1), 0.0
            )


def kernel(x_nchw, w_oihw, bias, gamma, beta):
    del bias
    N, C, H, W = x_nchw.shape
    Cout, _, KH, KW = w_oihw.shape
    assert KH == 3 and KW == 3

    P = H * W
    NS = N // _BATCH

    x_flat = x_nchw.reshape(N, C, P)
    wt = jnp.transpose(w_oihw, (2, 3, 0, 1)).reshape(9, Cout, C)
    wt = wt.astype(jnp.bfloat16)

    cparams = pltpu.CompilerParams(
        dimension_semantics=("arbitrary", "arbitrary"),
        vmem_limit_bytes=100 * 1024 * 1024,
    )

    out_flat = pl.pallas_call(
        functools.partial(_fused_kernel, h=H, w=W, n=N * P),
        out_shape=jax.ShapeDtypeStruct((N, Cout, P), jnp.float32),
        grid=(2, NS),
        in_specs=[
            pl.BlockSpec((_BATCH, C, P), lambda p, i: (i * (1 - p), 0, 0)),
            pl.BlockSpec((9, Cout, C), lambda p, i: (0, 0, 0)),
            pl.BlockSpec((Cout, 1), lambda p, i: (0, 0)),
            pl.BlockSpec((Cout, 1), lambda p, i: (0, 0)),
        ],
        out_specs=pl.BlockSpec((_BATCH, Cout, P), lambda p, i: (i * p, 0, 0)),
        scratch_shapes=[
            pltpu.VMEM((C, _ALIGN + P + _ALIGN), jnp.bfloat16),
            pltpu.VMEM((N, Cout, P), jnp.bfloat16),
            pltpu.VMEM((Cout, 2), jnp.float32),
            pltpu.VMEM((16, P), jnp.float32),
        ],
        compiler_params=cparams,
    )(x_flat, wt, gamma.astype(jnp.float32).reshape(Cout, 1),
      beta.astype(jnp.float32).reshape(Cout, 1))
    return out_flat.reshape(N, Cout, H, W)


# speedup vs baseline: 1.1659x; 1.0012x over previous
import functools

import jax
import jax.numpy as jnp
from jax.experimental import pallas as pl
from jax.experimental.pallas import tpu as pltpu

_BN_EPS = 1e-5
_ALIGN = 128
_BATCH = 4


def _fused_kernel(x_ref, w_ref, g_ref, b_ref, out_ref, xs_ref, ys_ref,
                  st_ref, mask_ref, *, h, w, n):
    n_pix = h * w
    halo = w + 1
    phase = pl.program_id(0)
    step = pl.program_id(1)

    @pl.when((phase == 0) & (step == 0))
    def _init_constants():
        cin = x_ref.shape[1]
        xs_ref[:, :_ALIGN] = jnp.zeros((cin, _ALIGN), xs_ref.dtype)
        xs_ref[:, _ALIGN + n_pix:] = jnp.zeros(
            (cin, xs_ref.shape[1] - _ALIGN - n_pix), xs_ref.dtype
        )
        col = jax.lax.broadcasted_iota(jnp.int32, (8, n_pix), 1) % w
        mask_ref[0:8] = (col > 0).astype(jnp.float32)
        mask_ref[8:16] = (col < w - 1).astype(jnp.float32)

    @pl.when(phase == 0)
    def _conv_phase():
        m0 = mask_ref[0:1]
        m2 = mask_ref[8:9]
        part = jnp.zeros((st_ref.shape[0], 2), jnp.float32)
        for j in range(_BATCH):
            xs_ref[:, _ALIGN:_ALIGN + n_pix] = x_ref[j].astype(xs_ref.dtype)
            xs = xs_ref[...]
            accs = []
            for kw in range(3):
                acc = jnp.zeros((ys_ref.shape[1], n_pix), jnp.float32)
                for kh in range(3):
                    off = _ALIGN - halo + kh * w + kw
                    acc = acc + jnp.dot(
                        w_ref[3 * kh + kw], xs[:, off:off + n_pix],
                        preferred_element_type=jnp.float32,
                    )
                accs.append(acc)
            acc = accs[1] + m0 * accs[0] + m2 * accs[2]
            ys_ref[step * _BATCH + j] = acc.astype(ys_ref.dtype)
            ssum = jnp.sum(acc, axis=1, keepdims=True)
            ssq = jnp.sum(acc * acc, axis=1, keepdims=True)
            part = part + jnp.concatenate([ssum, ssq], axis=1)

        @pl.when(step == 0)
        def _init():
            st_ref[...] = part

        @pl.when(step > 0)
        def _accum():
            st_ref[...] = st_ref[...] + part

    @pl.when(phase == 1)
    def _bn_phase():
        st = st_ref[...]
        mean = st[:, 0] / n
        var = jnp.maximum(st[:, 1] / n - mean * mean, 0.0)
        scale = g_ref[...][:, 0] * jax.lax.rsqrt(var + _BN_EPS)
        shift = b_ref[...][:, 0] - mean * scale
        for j in range(_BATCH):
            y = ys_ref[step * _BATCH + j].astype(jnp.float32)
            out_ref[j] = jnp.maximum(
                y * scale.reshape(-1, 1) + shift.reshape(-1, 1), 0.0
            )


def kernel(x_nchw, w_oihw, bias, gamma, beta):
    del bias
    N, C, H, W = x_nchw.shape
    Cout, _, KH, KW = w_oihw.shape
    assert KH == 3 and KW == 3

    P = H * W
    NS = N // _BATCH

    x_flat = x_nchw.reshape(N, C, P)
    wt = jnp.transpose(w_oihw, (2, 3, 0, 1)).reshape(9, Cout, C)
    wt = wt.astype(jnp.bfloat16)

    cparams = pltpu.CompilerParams(
        dimension_semantics=("arbitrary", "arbitrary"),
        vmem_limit_bytes=100 * 1024 * 1024,
    )

    out_flat = pl.pallas_call(
        functools.partial(_fused_kernel, h=H, w=W, n=N * P),
        out_shape=jax.ShapeDtypeStruct((N, Cout, P), jnp.float32),
        grid=(2, NS),
        in_specs=[
            pl.BlockSpec((_BATCH, C, P), lambda p, i: (i * (1 - p), 0, 0)),
            pl.BlockSpec((9, Cout, C), lambda p, i: (0, 0, 0)),
            pl.BlockSpec((Cout, 1), lambda p, i: (0, 0)),
            pl.BlockSpec((Cout, 1), lambda p, i: (0, 0)),
        ],
        out_specs=pl.BlockSpec((_BATCH, Cout, P), lambda p, i: (i * p, 0, 0)),
        scratch_shapes=[
            pltpu.VMEM((C, _ALIGN + P + _ALIGN), jnp.bfloat16),
            pltpu.VMEM((N, Cout, P), jnp.bfloat16),
            pltpu.VMEM((Cout, 2), jnp.float32),
            pltpu.VMEM((16, P), jnp.float32),
        ],
        compiler_params=cparams,
    )(x_flat, wt, gamma.astype(jnp.float32).reshape(Cout, 1),
      beta.astype(jnp.float32).reshape(Cout, 1))
    return out_flat.reshape(N, Cout, H, W)
